# baseline (speedup 1.0000x reference)
import functools

import jax
import jax.numpy as jnp
from jax import lax
from jax.experimental import pallas as pl
from jax.experimental.pallas import tpu as pltpu
from jax.experimental.pallas import tpu_sc as plsc

N, E, F_IN, H, R, C = 10000, 160000, 256, 64, 16, 7

NC, NS = 2, 16
NW = NC * NS
SUB = 128
CC = 4
CHUNK = SUB * CC
NCHUNK = 10
EPW = CHUNK * NCHUNK
E_PAD = EPW * NW
NPAD = 10240
ROWS_PER_TILE = NPAD // NS
ZROWS = 64


def _mm_body(a_ref, b_ref, o_ref):
    o_ref[...] = jnp.dot(a_ref[...], b_ref[...],
                         preferred_element_type=jnp.float32)


def _mm_small(a, b):
    m, k = a.shape
    k2, n = b.shape
    return pl.pallas_call(
        _mm_body,
        out_shape=jax.ShapeDtypeStruct((m, n), jnp.float32),
    )(a, b)


def _mm_big(a, b):
    m, k = a.shape
    _, n = b.shape
    bm = 400
    return pl.pallas_call(
        _mm_body,
        grid=(m // bm,),
        in_specs=[
            pl.BlockSpec((bm, k), lambda i: (i, 0)),
            pl.BlockSpec((k, n), lambda i: (0, 0)),
        ],
        out_specs=pl.BlockSpec((bm, n), lambda i: (i, 0)),
        out_shape=jax.ShapeDtypeStruct((m, n), jnp.float32),
    )(a, b)


def _mm_rel_body(x_ref, w_ref, o_ref):
    o_ref[...] = jnp.dot(x_ref[...], w_ref[0],
                         preferred_element_type=jnp.float32)


def _mm_rel(x, w3):
    bm = 400
    return pl.pallas_call(
        _mm_rel_body,
        grid=(N // bm, R),
        in_specs=[
            pl.BlockSpec((bm, F_IN), lambda i, r: (i, 0)),
            pl.BlockSpec((1, F_IN, H), lambda i, r: (r, 0, 0)),
        ],
        out_specs=pl.BlockSpec((bm, H), lambda i, r: (r * (N // bm) + i, 0)),
        out_shape=jax.ShapeDtypeStruct((R * N, H), jnp.float32),
    )(x, w3)


def _h1_body(p0_ref, p1_ref, xr_ref, b_ref, o_ref):
    o_ref[...] = p0_ref[...] + p1_ref[...] + xr_ref[...] + b_ref[...]


def _h1_combine(p0, p1, xroot, bias):
    bm = 1000
    return pl.pallas_call(
        _h1_body,
        grid=(N // bm,),
        in_specs=[
            pl.BlockSpec((bm, H), lambda i: (i, 0)),
            pl.BlockSpec((bm, H), lambda i: (i, 0)),
            pl.BlockSpec((bm, H), lambda i: (i, 0)),
            pl.BlockSpec((1, H), lambda i: (0, 0)),
        ],
        out_specs=pl.BlockSpec((bm, H), lambda i: (i, 0)),
        out_shape=jax.ShapeDtypeStruct((N, H), jnp.float32),
    )(p0, p1, xroot, bias)


def _final_body(a0_ref, a1_ref, h1_ref, xl_ref, wr_ref, wo_ref, wl_ref,
                wf_ref, gb_ref, lb_ref, fb_ref, o_ref):
    agg = a0_ref[...] + a1_ref[...]
    h2 = (jnp.dot(agg, wr_ref[...], preferred_element_type=jnp.float32)
          + jnp.dot(h1_ref[...], wo_ref[...], preferred_element_type=jnp.float32)
          + gb_ref[...])
    hid = xl_ref[...] + jnp.dot(h2, wl_ref[...],
                                preferred_element_type=jnp.float32) + lb_ref[...]
    hid = jnp.maximum(hid, 0.0)
    lg = jnp.dot(hid, wf_ref[...], preferred_element_type=jnp.float32) + fb_ref[...]
    m = jnp.max(lg, axis=1, keepdims=True)
    o_ref[...] = lg - m - jnp.log(jnp.sum(jnp.exp(lg - m), axis=1, keepdims=True))


def _final(a0, a1, h1, xlin, w_rel, w_root, w_lin, w_fc, gb, lb, fb):
    bm = 1000
    row = lambda i: (i, 0)
    fix = lambda i: (0, 0)
    return pl.pallas_call(
        _final_body,
        grid=(N // bm,),
        in_specs=[
            pl.BlockSpec((bm, H), row),
            pl.BlockSpec((bm, H), row),
            pl.BlockSpec((bm, H), row),
            pl.BlockSpec((bm, H), row),
            pl.BlockSpec((H, H), fix),
            pl.BlockSpec((H, H), fix),
            pl.BlockSpec((H, H), fix),
            pl.BlockSpec((H, C), fix),
            pl.BlockSpec((1, H), fix),
            pl.BlockSpec((1, H), fix),
            pl.BlockSpec((1, C), fix),
        ],
        out_specs=pl.BlockSpec((bm, C), row),
        out_shape=jax.ShapeDtypeStruct((N, C), jnp.float32),
    )(a0, a1, h1, xlin, w_rel, w_root, w_lin, w_fc, gb, lb, fb)


def _make_sc_scatter(table_rows, scaled):
    mesh = plsc.VectorSubcoreMesh(core_axis_name="c", subcore_axis_name="s",
                                  num_cores=NC, num_subcores=NS)
    scratch = [
        pltpu.VMEM((2, CC, SUB), jnp.int32),
        pltpu.VMEM((2, CC, SUB), jnp.int32),
        pltpu.VMEM((2, CHUNK, H), jnp.float32),
        pltpu.VMEM((ZROWS, H), jnp.float32),
        pltpu.VMEM_SHARED((NPAD, H), jnp.float32),
        pltpu.SemaphoreType.DMA,
        pltpu.SemaphoreType.DMA,
    ]
    if scaled:
        scratch.insert(2, pltpu.VMEM((2, CHUNK, 16), jnp.float32))

    def body(*refs):
        if scaled:
            (gidx_hbm, dst_hbm, norm_hbm, table_hbm, out_hbm,
             gidx_v, dst_v, norm_v, msg_v, zero_v, acc_sh, gsem, ssem) = refs
        else:
            (gidx_hbm, dst_hbm, table_hbm, out_hbm,
             gidx_v, dst_v, msg_v, zero_v, acc_sh, gsem, ssem) = refs
        cid = lax.axis_index("c")
        sid = lax.axis_index("s")
        wid = sid * NC + cid

        zvec = jnp.zeros((16,), jnp.float32)
        for q in range(H // 16):
            def zrow(r, _, q=q):
                zero_v[r, pl.ds(q * 16, 16)] = zvec
                return 0
            lax.fori_loop(0, ZROWS, zrow, 0)
        for k in range(ROWS_PER_TILE // ZROWS):
            pltpu.sync_copy(
                zero_v, acc_sh.at[pl.ds(sid * ROWS_PER_TILE + k * ZROWS, ZROWS)])
        plsc.subcore_barrier()

        def fire_gather(k):
            b = k & 1
            row0 = wid * (EPW // SUB) + k * CC
            pltpu.sync_copy(gidx_hbm.at[pl.ds(row0, CC)], gidx_v.at[b])
            pltpu.sync_copy(dst_hbm.at[pl.ds(row0, CC)], dst_v.at[b])
            if scaled:
                e0 = wid * EPW + k * CHUNK
                pltpu.sync_copy(norm_hbm.at[pl.ds(e0, CHUNK)], norm_v.at[b])
            return [pltpu.async_copy(table_hbm.at[gidx_v.at[b].at[j]],
                                     msg_v.at[b, pl.ds(j * SUB, SUB)], gsem)
                    for j in range(CC)]

        pend_g = {0: fire_gather(0)}
        pend_s = {}
        for k in range(NCHUNK):
            b = k & 1
            if k + 1 < NCHUNK:
                if k - 1 in pend_s:
                    for d in pend_s.pop(k - 1):
                        d.wait()
                pend_g[k + 1] = fire_gather(k + 1)
            for d in pend_g.pop(k):
                d.wait()
            if scaled:
                @plsc.parallel_loop(0, CHUNK, 1, unroll=4)
                def _(r, b=b):
                    m = norm_v[b, r, :]
                    for q in range(H // 16):
                        sl = pl.ds(q * 16, 16)
                        msg_v[b, r, sl] = msg_v[b, r, sl] * m
            pend_s[k] = [pltpu.async_copy(msg_v.at[b, pl.ds(j * SUB, SUB)],
                                          acc_sh.at[dst_v.at[b].at[j]],
                                          ssem, add=True)
                         for j in range(CC)]
        for k in sorted(pend_s):
            for d in pend_s[k]:
                d.wait()

        plsc.subcore_barrier()
        pltpu.sync_copy(
            acc_sh.at[pl.ds(sid * ROWS_PER_TILE, ROWS_PER_TILE)],
            out_hbm.at[cid, pl.ds(sid * ROWS_PER_TILE, ROWS_PER_TILE)])

    return pl.kernel(
        body,
        out_type=jax.ShapeDtypeStruct((NC, NPAD, H), jnp.float32),
        mesh=mesh,
        scratch_types=scratch,
        compiler_params=pltpu.CompilerParams(use_tc_tiling_on_sc=False),
    )


def kernel(x, edge_index, edge_norm, edge_type, seq_lengths, umask,
           nodal_attn, avec, basis, comp, root, rgcn_bias,
           gc_w_rel, gc_w_root, gc_bias, lin_w, lin_b, fc_w, fc_b):
    src = edge_index[0].astype(jnp.int32)
    dst = edge_index[1].astype(jnp.int32)
    et = edge_type.astype(jnp.int32)
    pad = E_PAD - E
    zpad_i = jnp.zeros((pad,), jnp.int32)
    gidx = jnp.concatenate([src * R + et, zpad_i]).reshape(E_PAD // SUB, SUB)
    src2 = jnp.concatenate([src, zpad_i]).reshape(E_PAD // SUB, SUB)
    dst2 = jnp.concatenate(
        [dst, jnp.full((pad,), NPAD - 1, jnp.int32)]).reshape(E_PAD // SUB, SUB)
    norm_p = jnp.concatenate([edge_norm, jnp.zeros((pad,), jnp.float32)])
    norm_b = jnp.broadcast_to(norm_p[:, None], (E_PAD, 16))

    B = basis.shape[0]
    basis2 = basis.reshape(B, F_IN * H)
    basis_p = jnp.concatenate(
        [basis2, jnp.zeros((32 - B, F_IN * H), jnp.float32)], axis=0)
    comp_p = jnp.concatenate(
        [comp, jnp.zeros((R, 32 - B), jnp.float32)], axis=1)
    wflat = _mm_small(comp_p, basis_p)
    wmat = wflat.reshape(R, F_IN, H).transpose(1, 0, 2).reshape(F_IN, R * H)

    wbig = jnp.concatenate([wmat, root, lin_w[:F_IN]], axis=1)
    big = _mm_big(x.astype(jnp.bfloat16),
                  wbig.astype(jnp.bfloat16))
    xrel = big[:, :R * H].reshape(N * R, H)
    xroot = big[:, R * H:R * H + H]
    xlin = big[:, R * H + H:]

    sc1 = _make_sc_scatter(N * R, scaled=True)
    parts = sc1(gidx, dst2, norm_b, xrel)[:, :N]
    h1 = _h1_combine(parts[0], parts[1], xroot,
                     rgcn_bias.reshape(1, H))

    sc2 = _make_sc_scatter(N, scaled=False)
    aparts = sc2(src2, dst2, h1)[:, :N]

    return _final(aparts[0], aparts[1], h1, xlin,
                  gc_w_rel, gc_w_root, lin_w[F_IN:], fc_w,
                  gc_bias.reshape(1, H), lin_b.reshape(1, H),
                  fc_b.reshape(1, C))


# speedup vs baseline: 1.0656x; 1.0656x over previous
import functools

import jax
import jax.numpy as jnp
from jax import lax
from jax.experimental import pallas as pl
from jax.experimental.pallas import tpu as pltpu
from jax.experimental.pallas import tpu_sc as plsc

N, E, F_IN, H, R, C = 10000, 160000, 256, 64, 16, 7

NC, NS = 2, 16
NW = NC * NS
SUB = 128
CC = 4
CHUNK = SUB * CC
NCHUNK = 10
EPW = CHUNK * NCHUNK
E_PAD = EPW * NW
NPAD = 10240
ROWS_PER_TILE = NPAD // NS
ZROWS = 64


def _mm_body(a_ref, b_ref, o_ref):
    o_ref[...] = jnp.dot(a_ref[...], b_ref[...],
                         preferred_element_type=jnp.float32)


def _mm_small(a, b):
    m, k = a.shape
    k2, n = b.shape
    return pl.pallas_call(
        _mm_body,
        out_shape=jax.ShapeDtypeStruct((m, n), jnp.float32),
    )(a, b)


def _mm_big(a, b):
    m, k = a.shape
    _, n = b.shape
    bm = 1000
    return pl.pallas_call(
        _mm_body,
        grid=(m // bm,),
        in_specs=[
            pl.BlockSpec((bm, k), lambda i: (i, 0)),
            pl.BlockSpec((k, n), lambda i: (0, 0)),
        ],
        out_specs=pl.BlockSpec((bm, n), lambda i: (i, 0)),
        out_shape=jax.ShapeDtypeStruct((m, n), jnp.float32),
    )(a, b)


def _mm_rel_body(x_ref, w_ref, o_ref):
    o_ref[...] = jnp.dot(x_ref[...], w_ref[0],
                         preferred_element_type=jnp.float32)


def _mm_rel(x, w3):
    bm = 400
    return pl.pallas_call(
        _mm_rel_body,
        grid=(N // bm, R),
        in_specs=[
            pl.BlockSpec((bm, F_IN), lambda i, r: (i, 0)),
            pl.BlockSpec((1, F_IN, H), lambda i, r: (r, 0, 0)),
        ],
        out_specs=pl.BlockSpec((bm, H), lambda i, r: (r * (N // bm) + i, 0)),
        out_shape=jax.ShapeDtypeStruct((R * N, H), jnp.float32),
    )(x, w3)


def _h1_body(p0_ref, p1_ref, xr_ref, b_ref, o_ref):
    o_ref[...] = p0_ref[...] + p1_ref[...] + xr_ref[...] + b_ref[...]


def _h1_combine(p0, p1, xroot, bias):
    bm = 1000
    return pl.pallas_call(
        _h1_body,
        grid=(N // bm,),
        in_specs=[
            pl.BlockSpec((bm, H), lambda i: (i, 0)),
            pl.BlockSpec((bm, H), lambda i: (i, 0)),
            pl.BlockSpec((bm, H), lambda i: (i, 0)),
            pl.BlockSpec((1, H), lambda i: (0, 0)),
        ],
        out_specs=pl.BlockSpec((bm, H), lambda i: (i, 0)),
        out_shape=jax.ShapeDtypeStruct((N, H), jnp.float32),
    )(p0, p1, xroot, bias)


def _final_body(a0_ref, a1_ref, h1_ref, xl_ref, wr_ref, wo_ref, wl_ref,
                wf_ref, gb_ref, lb_ref, fb_ref, o_ref):
    agg = a0_ref[...] + a1_ref[...]
    h2 = (jnp.dot(agg, wr_ref[...], preferred_element_type=jnp.float32)
          + jnp.dot(h1_ref[...], wo_ref[...], preferred_element_type=jnp.float32)
          + gb_ref[...])
    hid = xl_ref[...] + jnp.dot(h2, wl_ref[...],
                                preferred_element_type=jnp.float32) + lb_ref[...]
    hid = jnp.maximum(hid, 0.0)
    lg = jnp.dot(hid, wf_ref[...], preferred_element_type=jnp.float32) + fb_ref[...]
    m = jnp.max(lg, axis=1, keepdims=True)
    o_ref[...] = lg - m - jnp.log(jnp.sum(jnp.exp(lg - m), axis=1, keepdims=True))


def _final(a0, a1, h1, xlin, w_rel, w_root, w_lin, w_fc, gb, lb, fb):
    bm = 1000
    row = lambda i: (i, 0)
    fix = lambda i: (0, 0)
    return pl.pallas_call(
        _final_body,
        grid=(N // bm,),
        in_specs=[
            pl.BlockSpec((bm, H), row),
            pl.BlockSpec((bm, H), row),
            pl.BlockSpec((bm, H), row),
            pl.BlockSpec((bm, H), row),
            pl.BlockSpec((H, H), fix),
            pl.BlockSpec((H, H), fix),
            pl.BlockSpec((H, H), fix),
            pl.BlockSpec((H, C), fix),
            pl.BlockSpec((1, H), fix),
            pl.BlockSpec((1, H), fix),
            pl.BlockSpec((1, C), fix),
        ],
        out_specs=pl.BlockSpec((bm, C), row),
        out_shape=jax.ShapeDtypeStruct((N, C), jnp.float32),
    )(a0, a1, h1, xlin, w_rel, w_root, w_lin, w_fc, gb, lb, fb)


def _make_sc_scatter(table_rows, scaled):
    mesh = plsc.VectorSubcoreMesh(core_axis_name="c", subcore_axis_name="s",
                                  num_cores=NC, num_subcores=NS)
    scratch = [
        pltpu.VMEM((2, CC, SUB), jnp.int32),
        pltpu.VMEM((2, CC, SUB), jnp.int32),
        pltpu.VMEM((2, CHUNK, H), jnp.float32),
        pltpu.VMEM((ZROWS, H), jnp.float32),
        pltpu.VMEM_SHARED((NPAD, H), jnp.float32),
        pltpu.SemaphoreType.DMA,
        pltpu.SemaphoreType.DMA,
    ]
    if scaled:
        scratch.insert(2, pltpu.VMEM((2, CHUNK, 16), jnp.float32))

    def body(*refs):
        if scaled:
            (gidx_hbm, dst_hbm, norm_hbm, table_hbm, out_hbm,
             gidx_v, dst_v, norm_v, msg_v, zero_v, acc_sh, gsem, ssem) = refs
        else:
            (gidx_hbm, dst_hbm, table_hbm, out_hbm,
             gidx_v, dst_v, msg_v, zero_v, acc_sh, gsem, ssem) = refs
        cid = lax.axis_index("c")
        sid = lax.axis_index("s")
        wid = sid * NC + cid

        zvec = jnp.zeros((16,), jnp.float32)
        for q in range(H // 16):
            def zrow(r, _, q=q):
                zero_v[r, pl.ds(q * 16, 16)] = zvec
                return 0
            lax.fori_loop(0, ZROWS, zrow, 0)
        for k in range(ROWS_PER_TILE // ZROWS):
            pltpu.sync_copy(
                zero_v, acc_sh.at[pl.ds(sid * ROWS_PER_TILE + k * ZROWS, ZROWS)])
        plsc.subcore_barrier()

        def fire_gather(k):
            b = k & 1
            row0 = wid * (EPW // SUB) + k * CC
            pltpu.sync_copy(gidx_hbm.at[pl.ds(row0, CC)], gidx_v.at[b])
            pltpu.sync_copy(dst_hbm.at[pl.ds(row0, CC)], dst_v.at[b])
            if scaled:
                e0 = wid * EPW + k * CHUNK
                pltpu.sync_copy(norm_hbm.at[pl.ds(e0, CHUNK)], norm_v.at[b])
            return [pltpu.async_copy(table_hbm.at[gidx_v.at[b].at[j]],
                                     msg_v.at[b, pl.ds(j * SUB, SUB)], gsem)
                    for j in range(CC)]

        pend_g = {0: fire_gather(0)}
        pend_s = {}
        for k in range(NCHUNK):
            b = k & 1
            if k + 1 < NCHUNK:
                if k - 1 in pend_s:
                    for d in pend_s.pop(k - 1):
                        d.wait()
                pend_g[k + 1] = fire_gather(k + 1)
            for d in pend_g.pop(k):
                d.wait()
            if scaled:
                @plsc.parallel_loop(0, CHUNK, 1, unroll=4)
                def _(r, b=b):
                    m = norm_v[b, r, :]
                    for q in range(H // 16):
                        sl = pl.ds(q * 16, 16)
                        msg_v[b, r, sl] = msg_v[b, r, sl] * m
            pend_s[k] = [pltpu.async_copy(msg_v.at[b, pl.ds(j * SUB, SUB)],
                                          acc_sh.at[dst_v.at[b].at[j]],
                                          ssem, add=True)
                         for j in range(CC)]
        for k in sorted(pend_s):
            for d in pend_s[k]:
                d.wait()

        plsc.subcore_barrier()
        pltpu.sync_copy(
            acc_sh.at[pl.ds(sid * ROWS_PER_TILE, ROWS_PER_TILE)],
            out_hbm.at[cid, pl.ds(sid * ROWS_PER_TILE, ROWS_PER_TILE)])

    return pl.kernel(
        body,
        out_type=jax.ShapeDtypeStruct((NC, NPAD, H), jnp.float32),
        mesh=mesh,
        scratch_types=scratch,
        compiler_params=pltpu.CompilerParams(use_tc_tiling_on_sc=False),
    )


def kernel(x, edge_index, edge_norm, edge_type, seq_lengths, umask,
           nodal_attn, avec, basis, comp, root, rgcn_bias,
           gc_w_rel, gc_w_root, gc_bias, lin_w, lin_b, fc_w, fc_b):
    src = edge_index[0].astype(jnp.int32)
    dst = edge_index[1].astype(jnp.int32)
    et = edge_type.astype(jnp.int32)
    pad = E_PAD - E
    zpad_i = jnp.zeros((pad,), jnp.int32)
    gidx = jnp.concatenate([src * R + et, zpad_i]).reshape(E_PAD // SUB, SUB)
    src2 = jnp.concatenate([src, zpad_i]).reshape(E_PAD // SUB, SUB)
    dst2 = jnp.concatenate(
        [dst, jnp.full((pad,), NPAD - 1, jnp.int32)]).reshape(E_PAD // SUB, SUB)
    norm_p = jnp.concatenate([edge_norm, jnp.zeros((pad,), jnp.float32)])
    norm_b = jnp.broadcast_to(norm_p[:, None], (E_PAD, 16))

    B = basis.shape[0]
    basis2 = basis.reshape(B, F_IN * H)
    basis_p = jnp.concatenate(
        [basis2, jnp.zeros((32 - B, F_IN * H), jnp.float32)], axis=0)
    comp_p = jnp.concatenate(
        [comp, jnp.zeros((R, 32 - B), jnp.float32)], axis=1)
    wflat = _mm_small(comp_p, basis_p)
    wmat = wflat.reshape(R, F_IN, H).transpose(1, 0, 2).reshape(F_IN, R * H)

    wbig = jnp.concatenate([wmat, root, lin_w[:F_IN]], axis=1)
    big = _mm_big(x, wbig)
    xrel = big[:, :R * H].reshape(N * R, H)
    xroot = big[:, R * H:R * H + H]
    xlin = big[:, R * H + H:]

    sc1 = _make_sc_scatter(N * R, scaled=True)
    parts = sc1(gidx, dst2, norm_b, xrel)[:, :N]
    h1 = _h1_combine(parts[0], parts[1], xroot,
                     rgcn_bias.reshape(1, H))

    sc2 = _make_sc_scatter(N, scaled=False)
    aparts = sc2(src2, dst2, h1)[:, :N]

    return _final(aparts[0], aparts[1], h1, xlin,
                  gc_w_rel, gc_w_root, lin_w[F_IN:], fc_w,
                  gc_bias.reshape(1, H), lin_b.reshape(1, H),
                  fc_b.reshape(1, C))


# speedup vs baseline: 1.0662x; 1.0006x over previous
import functools

import jax
import jax.numpy as jnp
from jax import lax
from jax.experimental import pallas as pl
from jax.experimental.pallas import tpu as pltpu
from jax.experimental.pallas import tpu_sc as plsc

N, E, F_IN, H, R, C = 10000, 160000, 256, 64, 16, 7

NC, NS = 2, 16
NW = NC * NS
SUB = 128
CC = 4
CHUNK = SUB * CC
NCHUNK = 10
EPW = CHUNK * NCHUNK
E_PAD = EPW * NW
NPAD = 10240
ROWS_PER_TILE = NPAD // NS
ZROWS = 64


def _mm_body(a_ref, b_ref, o_ref):
    o_ref[...] = jnp.dot(a_ref[...], b_ref[...],
                         preferred_element_type=jnp.float32)


def _mm_body_bf16(a_ref, b_ref, o_ref):
    o_ref[...] = jnp.dot(a_ref[...].astype(jnp.bfloat16),
                         b_ref[...].astype(jnp.bfloat16),
                         preferred_element_type=jnp.float32)


def _mm_small(a, b):
    m, k = a.shape
    k2, n = b.shape
    return pl.pallas_call(
        _mm_body,
        out_shape=jax.ShapeDtypeStruct((m, n), jnp.float32),
    )(a, b)


def _mm_big(a, b):
    m, k = a.shape
    _, n = b.shape
    bm = 1000
    return pl.pallas_call(
        _mm_body_bf16,
        grid=(m // bm,),
        in_specs=[
            pl.BlockSpec((bm, k), lambda i: (i, 0)),
            pl.BlockSpec((k, n), lambda i: (0, 0)),
        ],
        out_specs=pl.BlockSpec((bm, n), lambda i: (i, 0)),
        out_shape=jax.ShapeDtypeStruct((m, n), jnp.float32),
    )(a, b)


def _mm_rel_body(x_ref, w_ref, o_ref):
    o_ref[...] = jnp.dot(x_ref[...], w_ref[0],
                         preferred_element_type=jnp.float32)


def _mm_rel(x, w3):
    bm = 400
    return pl.pallas_call(
        _mm_rel_body,
        grid=(N // bm, R),
        in_specs=[
            pl.BlockSpec((bm, F_IN), lambda i, r: (i, 0)),
            pl.BlockSpec((1, F_IN, H), lambda i, r: (r, 0, 0)),
        ],
        out_specs=pl.BlockSpec((bm, H), lambda i, r: (r * (N // bm) + i, 0)),
        out_shape=jax.ShapeDtypeStruct((R * N, H), jnp.float32),
    )(x, w3)


def _h1_body(p0_ref, p1_ref, xr_ref, b_ref, o_ref):
    o_ref[...] = p0_ref[...] + p1_ref[...] + xr_ref[...] + b_ref[...]


def _h1_combine(p0, p1, xroot, bias):
    bm = 1000
    return pl.pallas_call(
        _h1_body,
        grid=(N // bm,),
        in_specs=[
            pl.BlockSpec((bm, H), lambda i: (i, 0)),
            pl.BlockSpec((bm, H), lambda i: (i, 0)),
            pl.BlockSpec((bm, H), lambda i: (i, 0)),
            pl.BlockSpec((1, H), lambda i: (0, 0)),
        ],
        out_specs=pl.BlockSpec((bm, H), lambda i: (i, 0)),
        out_shape=jax.ShapeDtypeStruct((N, H), jnp.float32),
    )(p0, p1, xroot, bias)


def _final_body(a0_ref, a1_ref, h1_ref, xl_ref, wr_ref, wo_ref, wl_ref,
                wf_ref, gb_ref, lb_ref, fb_ref, o_ref):
    agg = a0_ref[...] + a1_ref[...]
    h2 = (jnp.dot(agg, wr_ref[...], preferred_element_type=jnp.float32)
          + jnp.dot(h1_ref[...], wo_ref[...], preferred_element_type=jnp.float32)
          + gb_ref[...])
    hid = xl_ref[...] + jnp.dot(h2, wl_ref[...],
                                preferred_element_type=jnp.float32) + lb_ref[...]
    hid = jnp.maximum(hid, 0.0)
    lg = jnp.dot(hid, wf_ref[...], preferred_element_type=jnp.float32) + fb_ref[...]
    m = jnp.max(lg, axis=1, keepdims=True)
    o_ref[...] = lg - m - jnp.log(jnp.sum(jnp.exp(lg - m), axis=1, keepdims=True))


def _final(a0, a1, h1, xlin, w_rel, w_root, w_lin, w_fc, gb, lb, fb):
    bm = 1000
    row = lambda i: (i, 0)
    fix = lambda i: (0, 0)
    return pl.pallas_call(
        _final_body,
        grid=(N // bm,),
        in_specs=[
            pl.BlockSpec((bm, H), row),
            pl.BlockSpec((bm, H), row),
            pl.BlockSpec((bm, H), row),
            pl.BlockSpec((bm, H), row),
            pl.BlockSpec((H, H), fix),
            pl.BlockSpec((H, H), fix),
            pl.BlockSpec((H, H), fix),
            pl.BlockSpec((H, C), fix),
            pl.BlockSpec((1, H), fix),
            pl.BlockSpec((1, H), fix),
            pl.BlockSpec((1, C), fix),
        ],
        out_specs=pl.BlockSpec((bm, C), row),
        out_shape=jax.ShapeDtypeStruct((N, C), jnp.float32),
    )(a0, a1, h1, xlin, w_rel, w_root, w_lin, w_fc, gb, lb, fb)


def _make_sc_scatter(table_rows, scaled):
    mesh = plsc.VectorSubcoreMesh(core_axis_name="c", subcore_axis_name="s",
                                  num_cores=NC, num_subcores=NS)
    scratch = [
        pltpu.VMEM((2, CC, SUB), jnp.int32),
        pltpu.VMEM((2, CC, SUB), jnp.int32),
        pltpu.VMEM((2, CHUNK, H), jnp.float32),
        pltpu.VMEM((ZROWS, H), jnp.float32),
        pltpu.VMEM_SHARED((NPAD, H), jnp.float32),
        pltpu.SemaphoreType.DMA,
        pltpu.SemaphoreType.DMA,
    ]
    if scaled:
        scratch.insert(2, pltpu.VMEM((2, CHUNK, 16), jnp.float32))

    def body(*refs):
        if scaled:
            (gidx_hbm, dst_hbm, norm_hbm, table_hbm, out_hbm,
             gidx_v, dst_v, norm_v, msg_v, zero_v, acc_sh, gsem, ssem) = refs
        else:
            (gidx_hbm, dst_hbm, table_hbm, out_hbm,
             gidx_v, dst_v, msg_v, zero_v, acc_sh, gsem, ssem) = refs
        cid = lax.axis_index("c")
        sid = lax.axis_index("s")
        wid = sid * NC + cid

        zvec = jnp.zeros((16,), jnp.float32)
        for q in range(H // 16):
            def zrow(r, _, q=q):
                zero_v[r, pl.ds(q * 16, 16)] = zvec
                return 0
            lax.fori_loop(0, ZROWS, zrow, 0)
        for k in range(ROWS_PER_TILE // ZROWS):
            pltpu.sync_copy(
                zero_v, acc_sh.at[pl.ds(sid * ROWS_PER_TILE + k * ZROWS, ZROWS)])
        plsc.subcore_barrier()

        def fire_gather(k):
            b = k & 1
            row0 = wid * (EPW // SUB) + k * CC
            pltpu.sync_copy(gidx_hbm.at[pl.ds(row0, CC)], gidx_v.at[b])
            pltpu.sync_copy(dst_hbm.at[pl.ds(row0, CC)], dst_v.at[b])
            if scaled:
                e0 = wid * EPW + k * CHUNK
                pltpu.sync_copy(norm_hbm.at[pl.ds(e0, CHUNK)], norm_v.at[b])
            return [pltpu.async_copy(table_hbm.at[gidx_v.at[b].at[j]],
                                     msg_v.at[b, pl.ds(j * SUB, SUB)], gsem)
                    for j in range(CC)]

        pend_g = {0: fire_gather(0)}
        pend_s = {}
        for k in range(NCHUNK):
            b = k & 1
            if k + 1 < NCHUNK:
                if k - 1 in pend_s:
                    for d in pend_s.pop(k - 1):
                        d.wait()
                pend_g[k + 1] = fire_gather(k + 1)
            for d in pend_g.pop(k):
                d.wait()
            if scaled:
                @plsc.parallel_loop(0, CHUNK, 1, unroll=4)
                def _(r, b=b):
                    m = norm_v[b, r, :]
                    for q in range(H // 16):
                        sl = pl.ds(q * 16, 16)
                        msg_v[b, r, sl] = msg_v[b, r, sl] * m
            pend_s[k] = [pltpu.async_copy(msg_v.at[b, pl.ds(j * SUB, SUB)],
                                          acc_sh.at[dst_v.at[b].at[j]],
                                          ssem, add=True)
                         for j in range(CC)]
        for k in sorted(pend_s):
            for d in pend_s[k]:
                d.wait()

        plsc.subcore_barrier()
        pltpu.sync_copy(
            acc_sh.at[pl.ds(sid * ROWS_PER_TILE, ROWS_PER_TILE)],
            out_hbm.at[cid, pl.ds(sid * ROWS_PER_TILE, ROWS_PER_TILE)])

    return pl.kernel(
        body,
        out_type=jax.ShapeDtypeStruct((NC, NPAD, H), jnp.float32),
        mesh=mesh,
        scratch_types=scratch,
        compiler_params=pltpu.CompilerParams(use_tc_tiling_on_sc=False),
    )


def kernel(x, edge_index, edge_norm, edge_type, seq_lengths, umask,
           nodal_attn, avec, basis, comp, root, rgcn_bias,
           gc_w_rel, gc_w_root, gc_bias, lin_w, lin_b, fc_w, fc_b):
    src = edge_index[0].astype(jnp.int32)
    dst = edge_index[1].astype(jnp.int32)
    et = edge_type.astype(jnp.int32)
    pad = E_PAD - E
    zpad_i = jnp.zeros((pad,), jnp.int32)
    gidx = jnp.concatenate([src * R + et, zpad_i]).reshape(E_PAD // SUB, SUB)
    src2 = jnp.concatenate([src, zpad_i]).reshape(E_PAD // SUB, SUB)
    dst2 = jnp.concatenate(
        [dst, jnp.full((pad,), NPAD - 1, jnp.int32)]).reshape(E_PAD // SUB, SUB)
    norm_p = jnp.concatenate([edge_norm, jnp.zeros((pad,), jnp.float32)])
    norm_b = jnp.broadcast_to(norm_p[:, None], (E_PAD, 16))

    B = basis.shape[0]
    basis2 = basis.reshape(B, F_IN * H)
    basis_p = jnp.concatenate(
        [basis2, jnp.zeros((32 - B, F_IN * H), jnp.float32)], axis=0)
    comp_p = jnp.concatenate(
        [comp, jnp.zeros((R, 32 - B), jnp.float32)], axis=1)
    wflat = _mm_small(comp_p, basis_p)
    wmat = wflat.reshape(R, F_IN, H).transpose(1, 0, 2).reshape(F_IN, R * H)

    wbig = jnp.concatenate([wmat, root, lin_w[:F_IN]], axis=1)
    big = _mm_big(x, wbig)
    xrel = big[:, :R * H].reshape(N * R, H)
    xroot = big[:, R * H:R * H + H]
    xlin = big[:, R * H + H:]

    sc1 = _make_sc_scatter(N * R, scaled=True)
    parts = sc1(gidx, dst2, norm_b, xrel)[:, :N]
    h1 = _h1_combine(parts[0], parts[1], xroot,
                     rgcn_bias.reshape(1, H))

    sc2 = _make_sc_scatter(N, scaled=False)
    aparts = sc2(src2, dst2, h1)[:, :N]

    return _final(aparts[0], aparts[1], h1, xlin,
                  gc_w_rel, gc_w_root, lin_w[F_IN:], fc_w,
                  gc_bias.reshape(1, H), lin_b.reshape(1, H),
                  fc_b.reshape(1, C))


# speedup vs baseline: 1.0669x; 1.0007x over previous
import jax
import jax.numpy as jnp
from jax import lax
from jax.experimental import pallas as pl
from jax.experimental.pallas import tpu as pltpu
from jax.experimental.pallas import tpu_sc as plsc

N, E, F_IN, H, R, C = 10000, 160000, 256, 64, 16, 7

NC, NS = 2, 16
NW = NC * NS
SUB = 128
CC = 4
CHUNK = SUB * CC
NCHUNK = 10
EPW = CHUNK * NCHUNK
E_PAD = EPW * NW
NPAD = 10240
ROWS_PER_TILE = NPAD // NS
ZROWS = 64


def _mm_body(a_ref, b_ref, o_ref):
    o_ref[...] = jnp.dot(a_ref[...], b_ref[...],
                         preferred_element_type=jnp.float32)


def _mm_small(a, b):
    m, k = a.shape
    k2, n = b.shape
    return pl.pallas_call(
        _mm_body,
        out_shape=jax.ShapeDtypeStruct((m, n), jnp.float32),
    )(a, b)


def _mm_big(a, b):
    m, k = a.shape
    _, n = b.shape
    bm = 1000
    return pl.pallas_call(
        _mm_body,
        grid=(m // bm,),
        in_specs=[
            pl.BlockSpec((bm, k), lambda i: (i, 0)),
            pl.BlockSpec((k, n), lambda i: (0, 0)),
        ],
        out_specs=pl.BlockSpec((bm, n), lambda i: (i, 0)),
        out_shape=jax.ShapeDtypeStruct((m, n), jnp.float32),
    )(a, b)


def _h1_body(p0_ref, p1_ref, xr_ref, b_ref, o_ref):
    o_ref[...] = p0_ref[...] + p1_ref[...] + xr_ref[...] + b_ref[...]


def _h1_combine(p0, p1, xroot, bias):
    bm = 1000
    return pl.pallas_call(
        _h1_body,
        grid=(N // bm,),
        in_specs=[
            pl.BlockSpec((bm, H), lambda i: (i, 0)),
            pl.BlockSpec((bm, H), lambda i: (i, 0)),
            pl.BlockSpec((bm, H), lambda i: (i, 0)),
            pl.BlockSpec((1, H), lambda i: (0, 0)),
        ],
        out_specs=pl.BlockSpec((bm, H), lambda i: (i, 0)),
        out_shape=jax.ShapeDtypeStruct((N, H), jnp.float32),
    )(p0, p1, xroot, bias)


def _final_body(a0_ref, a1_ref, h1_ref, xl_ref, wr_ref, wo_ref, wl_ref,
                wf_ref, gb_ref, lb_ref, fb_ref, o_ref):
    agg = a0_ref[...] + a1_ref[...]
    h2 = (jnp.dot(agg, wr_ref[...], preferred_element_type=jnp.float32)
          + jnp.dot(h1_ref[...], wo_ref[...], preferred_element_type=jnp.float32)
          + gb_ref[...])
    hid = xl_ref[...] + jnp.dot(h2, wl_ref[...],
                                preferred_element_type=jnp.float32) + lb_ref[...]
    hid = jnp.maximum(hid, 0.0)
    lg = jnp.dot(hid, wf_ref[...], preferred_element_type=jnp.float32) + fb_ref[...]
    m = jnp.max(lg, axis=1, keepdims=True)
    o_ref[...] = lg - m - jnp.log(jnp.sum(jnp.exp(lg - m), axis=1, keepdims=True))


def _final(a0, a1, h1, xlin, w_rel, w_root, w_lin, w_fc, gb, lb, fb):
    bm = 1000
    row = lambda i: (i, 0)
    fix = lambda i: (0, 0)
    return pl.pallas_call(
        _final_body,
        grid=(N // bm,),
        in_specs=[
            pl.BlockSpec((bm, H), row),
            pl.BlockSpec((bm, H), row),
            pl.BlockSpec((bm, H), row),
            pl.BlockSpec((bm, H), row),
            pl.BlockSpec((H, H), fix),
            pl.BlockSpec((H, H), fix),
            pl.BlockSpec((H, H), fix),
            pl.BlockSpec((H, C), fix),
            pl.BlockSpec((1, H), fix),
            pl.BlockSpec((1, H), fix),
            pl.BlockSpec((1, C), fix),
        ],
        out_specs=pl.BlockSpec((bm, C), row),
        out_shape=jax.ShapeDtypeStruct((N, C), jnp.float32),
    )(a0, a1, h1, xlin, w_rel, w_root, w_lin, w_fc, gb, lb, fb)


def _make_sc_scatter(table_rows, scaled):
    mesh = plsc.VectorSubcoreMesh(core_axis_name="c", subcore_axis_name="s",
                                  num_cores=NC, num_subcores=NS)
    scratch = [
        pltpu.VMEM((2, CC, SUB), jnp.int32),
        pltpu.VMEM((2, CC, SUB), jnp.int32),
        pltpu.VMEM((2, CHUNK, H), jnp.float32),
        pltpu.VMEM((ZROWS, H), jnp.float32),
        pltpu.VMEM_SHARED((NPAD, H), jnp.float32),
        pltpu.SemaphoreType.DMA,
        pltpu.SemaphoreType.DMA,
    ]
    if scaled:
        scratch.insert(2, pltpu.VMEM((2, CHUNK, 16), jnp.float32))

    def body(*refs):
        if scaled:
            (gidx_hbm, dst_hbm, norm_hbm, table_hbm, out_hbm,
             gidx_v, dst_v, norm_v, msg_v, zero_v, acc_sh, gsem, ssem) = refs
        else:
            (gidx_hbm, dst_hbm, table_hbm, out_hbm,
             gidx_v, dst_v, msg_v, zero_v, acc_sh, gsem, ssem) = refs
        cid = lax.axis_index("c")
        sid = lax.axis_index("s")
        wid = sid * NC + cid

        zvec = jnp.zeros((16,), jnp.float32)
        for q in range(H // 16):
            def zrow(r, _, q=q):
                zero_v[r, pl.ds(q * 16, 16)] = zvec
                return 0
            lax.fori_loop(0, ZROWS, zrow, 0)
        for k in range(ROWS_PER_TILE // ZROWS):
            pltpu.sync_copy(
                zero_v, acc_sh.at[pl.ds(sid * ROWS_PER_TILE + k * ZROWS, ZROWS)])
        plsc.subcore_barrier()

        def fire_gather(k):
            b = k & 1
            row0 = wid * (EPW // SUB) + k * CC
            pltpu.sync_copy(gidx_hbm.at[pl.ds(row0, CC)], gidx_v.at[b])
            pltpu.sync_copy(dst_hbm.at[pl.ds(row0, CC)], dst_v.at[b])
            if scaled:
                e0 = wid * EPW + k * CHUNK
                pltpu.sync_copy(norm_hbm.at[pl.ds(e0, CHUNK)], norm_v.at[b])
            return [pltpu.async_copy(table_hbm.at[gidx_v.at[b].at[j]],
                                     msg_v.at[b, pl.ds(j * SUB, SUB)], gsem)
                    for j in range(CC)]

        pend_g = {0: fire_gather(0)}
        pend_s = {}
        for k in range(NCHUNK):
            b = k & 1
            if k + 1 < NCHUNK:
                if k - 1 in pend_s:
                    for d in pend_s.pop(k - 1):
                        d.wait()
                pend_g[k + 1] = fire_gather(k + 1)
            for d in pend_g.pop(k):
                d.wait()
            if scaled:
                @plsc.parallel_loop(0, CHUNK, 1, unroll=4)
                def _(r, b=b):
                    m = norm_v[b, r, :]
                    for q in range(H // 16):
                        sl = pl.ds(q * 16, 16)
                        msg_v[b, r, sl] = msg_v[b, r, sl] * m
            pend_s[k] = [pltpu.async_copy(msg_v.at[b, pl.ds(j * SUB, SUB)],
                                          acc_sh.at[dst_v.at[b].at[j]],
                                          ssem, add=True)
                         for j in range(CC)]
        for k in sorted(pend_s):
            for d in pend_s[k]:
                d.wait()

        plsc.subcore_barrier()
        pltpu.sync_copy(
            acc_sh.at[pl.ds(sid * ROWS_PER_TILE, ROWS_PER_TILE)],
            out_hbm.at[cid, pl.ds(sid * ROWS_PER_TILE, ROWS_PER_TILE)])

    return pl.kernel(
        body,
        out_type=jax.ShapeDtypeStruct((NC, NPAD, H), jnp.float32),
        mesh=mesh,
        scratch_types=scratch,
        compiler_params=pltpu.CompilerParams(use_tc_tiling_on_sc=False),
    )


def kernel(x, edge_index, edge_norm, edge_type, seq_lengths, umask,
           nodal_attn, avec, basis, comp, root, rgcn_bias,
           gc_w_rel, gc_w_root, gc_bias, lin_w, lin_b, fc_w, fc_b):
    src = edge_index[0].astype(jnp.int32)
    dst = edge_index[1].astype(jnp.int32)
    et = edge_type.astype(jnp.int32)
    pad = E_PAD - E
    zpad_i = jnp.zeros((pad,), jnp.int32)
    gidx = jnp.concatenate([src * R + et, zpad_i]).reshape(E_PAD // SUB, SUB)
    src2 = jnp.concatenate([src, zpad_i]).reshape(E_PAD // SUB, SUB)
    dst2 = jnp.concatenate(
        [dst, jnp.full((pad,), NPAD - 1, jnp.int32)]).reshape(E_PAD // SUB, SUB)
    norm_p = jnp.concatenate([edge_norm, jnp.zeros((pad,), jnp.float32)])
    norm_b = jnp.broadcast_to(norm_p[:, None], (E_PAD, 16))

    B = basis.shape[0]
    basis2 = basis.reshape(B, F_IN * H)
    basis_p = jnp.concatenate(
        [basis2, jnp.zeros((32 - B, F_IN * H), jnp.float32)], axis=0)
    comp_p = jnp.concatenate(
        [comp, jnp.zeros((R, 32 - B), jnp.float32)], axis=1)
    wflat = _mm_small(comp_p, basis_p)
    wmat = wflat.reshape(R, F_IN, H).transpose(1, 0, 2).reshape(F_IN, R * H)

    wbig = jnp.concatenate([wmat, root, lin_w[:F_IN]], axis=1)
    big = _mm_big(x, wbig)
    xrel = big[:, :R * H].reshape(N * R, H)
    xroot = big[:, R * H:R * H + H]
    xlin = big[:, R * H + H:]

    sc1 = _make_sc_scatter(N * R, scaled=True)
    parts = sc1(gidx, dst2, norm_b, xrel)[:, :N]
    h1 = _h1_combine(parts[0], parts[1], xroot,
                     rgcn_bias.reshape(1, H))

    sc2 = _make_sc_scatter(N, scaled=False)
    aparts = sc2(src2, dst2, h1)[:, :N]

    return _final(aparts[0], aparts[1], h1, xlin,
                  gc_w_rel, gc_w_root, lin_w[F_IN:], fc_w,
                  gc_bias.reshape(1, H), lin_b.reshape(1, H),
                  fc_b.reshape(1, C))
